# baseline (speedup 1.0000x reference)
import functools

import jax
import jax.numpy as jnp
from jax import lax
from jax.experimental import pallas as pl
from jax.experimental.pallas import tpu as pltpu
from jax.experimental.pallas import tpu_sc as plsc

BATCH = 4
MAXLEN = 8192
EMBED_DIM = 768
LANES = 16
NUM_CORES = 2
NUM_SUBCORES = 16
NUM_WORKERS = NUM_CORES * NUM_SUBCORES
SEQ_PER_W = MAXLEN // NUM_WORKERS
ROWS = 64
NSUB = SEQ_PER_W // ROWS
COLS16 = EMBED_DIM // LANES


def _body(x_hbm, pos_hbm, out_hbm, pos_buf, x_buf):
    c = lax.axis_index("c")
    s = lax.axis_index("s")
    w = s * NUM_CORES + c
    base = w * SEQ_PER_W

    def sub(j, carry):
        row0 = base + j * ROWS
        pltpu.sync_copy(pos_hbm.at[pl.ds(row0, ROWS)], pos_buf)

        def bat(b, carry2):
            off = b * MAXLEN + row0
            pltpu.sync_copy(x_hbm.at[pl.ds(off, ROWS)], x_buf)

            def rowf(r, carry3):
                for cc in range(COLS16):
                    sl = pl.ds(cc * LANES, LANES)
                    x_buf[r, sl] = x_buf[r, sl] + pos_buf[r, sl]
                return carry3

            lax.fori_loop(0, ROWS, rowf, 0)
            pltpu.sync_copy(x_buf, out_hbm.at[pl.ds(off, ROWS)])
            return carry2

        lax.fori_loop(0, BATCH, bat, 0)
        return carry

    lax.fori_loop(0, NSUB, sub, 0)


@functools.partial(jax.jit)
def kernel(x, pos_table):
    mesh = plsc.VectorSubcoreMesh(
        core_axis_name="c", subcore_axis_name="s",
        num_cores=NUM_CORES, num_subcores=NUM_SUBCORES,
    )
    run = pl.kernel(
        _body,
        out_type=jax.ShapeDtypeStruct((BATCH * MAXLEN, EMBED_DIM), jnp.float32),
        mesh=mesh,
        scratch_types=[
            pltpu.VMEM((ROWS, EMBED_DIM), jnp.float32),
            pltpu.VMEM((ROWS, EMBED_DIM), jnp.float32),
        ],
    )
    out = run(x.reshape(BATCH * MAXLEN, EMBED_DIM), pos_table)
    return out.reshape(BATCH, MAXLEN, EMBED_DIM)


# speedup vs baseline: 1.3588x; 1.3588x over previous
import functools

import jax
import jax.numpy as jnp
from jax import lax
from jax.experimental import pallas as pl
from jax.experimental.pallas import tpu as pltpu
from jax.experimental.pallas import tpu_sc as plsc

BATCH = 4
MAXLEN = 8192
EMBED_DIM = 768
LANES = 16
NUM_CORES = 2
NUM_SUBCORES = 16
NUM_WORKERS = NUM_CORES * NUM_SUBCORES
SEQ_PER_W = MAXLEN // NUM_WORKERS
ROWS = 16
NSUB = SEQ_PER_W // ROWS
COLS16 = EMBED_DIM // LANES
NJJ = NSUB // 2


def _body(x_hbm, pos_hbm, out_hbm,
          x_b0, x_b1, x_b2, x_b3, p_b0, p_b1,
          sem_in, sem_out, sem_pos):
    c = lax.axis_index("c")
    s = lax.axis_index("s")
    w = s * NUM_CORES + c
    base = w * SEQ_PER_W
    x_bufs = (x_b0, x_b1, x_b2, x_b3)
    p_bufs = (p_b0, p_b1)

    def start_in(j, b, slot):
        off = b * MAXLEN + base + j * ROWS
        pltpu.async_copy(x_hbm.at[pl.ds(off, ROWS)], x_bufs[slot],
                         sem_in.at[slot])

    def wait_in(slot):
        pltpu.make_async_copy(x_hbm.at[pl.ds(0, ROWS)], x_bufs[slot],
                              sem_in.at[slot]).wait()

    def start_out(j, b, slot):
        off = b * MAXLEN + base + j * ROWS
        pltpu.async_copy(x_bufs[slot], out_hbm.at[pl.ds(off, ROWS)],
                         sem_out.at[slot])

    def wait_out(slot):
        pltpu.make_async_copy(x_bufs[slot], out_hbm.at[pl.ds(0, ROWS)],
                              sem_out.at[slot]).wait()

    def start_pos(j, pslot):
        pltpu.async_copy(pos_hbm.at[pl.ds(base + j * ROWS, ROWS)],
                         p_bufs[pslot], sem_pos.at[pslot])

    def wait_pos(pslot):
        pltpu.make_async_copy(pos_hbm.at[pl.ds(0, ROWS)], p_bufs[pslot],
                              sem_pos.at[pslot]).wait()

    def compute(slot, pslot):
        xb = x_bufs[slot]
        pb = p_bufs[pslot]

        def rowf(r, carry):
            for cc in range(COLS16):
                sl = pl.ds(cc * LANES, LANES)
                xb[r, sl] = xb[r, sl] + pb[r, sl]
            return carry

        lax.fori_loop(0, ROWS, rowf, 0)

    start_pos(0, 0)
    start_in(0, 0, 0)
    start_in(0, 1, 1)

    def outer(jj, carry):
        for pj in range(2):
            j = 2 * jj + pj
            for b in range(BATCH):
                slot = b
                nslot = (b + 2) % 4
                if pj == 0 and b < 2:
                    @pl.when(jj > 0)
                    def _():
                        wait_out(nslot)
                else:
                    wait_out(nslot)
                nb = (b + 2) % 4
                nj_shift = 1 if b >= 2 else 0
                if pj == 1 and b >= 2:
                    @pl.when(jj < NJJ - 1)
                    def _():
                        start_in(j + nj_shift, nb, nslot)
                else:
                    start_in(j + nj_shift, nb, nslot)
                if b == 0:
                    wait_pos(pj)
                    if pj == 0:
                        start_pos(j + 1, 1)
                    else:
                        @pl.when(jj < NJJ - 1)
                        def _():
                            start_pos(j + 1, 0)
                wait_in(slot)
                compute(slot, pj)
                start_out(j, b, slot)
        return carry

    lax.fori_loop(0, NJJ, outer, 0)
    wait_out(2)
    wait_out(3)


@functools.partial(jax.jit)
def kernel(x, pos_table):
    mesh = plsc.VectorSubcoreMesh(
        core_axis_name="c", subcore_axis_name="s",
        num_cores=NUM_CORES, num_subcores=NUM_SUBCORES,
    )
    run = pl.kernel(
        _body,
        out_type=jax.ShapeDtypeStruct((BATCH * MAXLEN, EMBED_DIM), jnp.float32),
        mesh=mesh,
        scratch_types=[
            pltpu.VMEM((ROWS, EMBED_DIM), jnp.float32),
            pltpu.VMEM((ROWS, EMBED_DIM), jnp.float32),
            pltpu.VMEM((ROWS, EMBED_DIM), jnp.float32),
            pltpu.VMEM((ROWS, EMBED_DIM), jnp.float32),
            pltpu.VMEM((ROWS, EMBED_DIM), jnp.float32),
            pltpu.VMEM((ROWS, EMBED_DIM), jnp.float32),
            pltpu.SemaphoreType.DMA((4,)),
            pltpu.SemaphoreType.DMA((4,)),
            pltpu.SemaphoreType.DMA((2,)),
        ],
    )
    out = run(x.reshape(BATCH * MAXLEN, EMBED_DIM), pos_table)
    return out.reshape(BATCH, MAXLEN, EMBED_DIM)


# speedup vs baseline: 1.5036x; 1.1066x over previous
import functools

import jax
import jax.numpy as jnp
from jax import lax
from jax.experimental import pallas as pl
from jax.experimental.pallas import tpu as pltpu
from jax.experimental.pallas import tpu_sc as plsc

BATCH = 4
MAXLEN = 8192
EMBED_DIM = 768
LANES = 16
NUM_CORES = 2
NUM_SUBCORES = 16
NUM_WORKERS = NUM_CORES * NUM_SUBCORES
SEQ_PER_W = MAXLEN // NUM_WORKERS
ROWS = 16
NSUB = SEQ_PER_W // ROWS
COLS16 = EMBED_DIM // LANES
NJJ = NSUB // 2


def _body(x_hbm, pos_hbm, out_hbm,
          x_b0, x_b1, x_b2, x_b3, x_b4, x_b5, x_b6, x_b7, p_b0, p_b1,
          sem_in, sem_out, sem_pos):
    c = lax.axis_index("c")
    s = lax.axis_index("s")
    w = s * NUM_CORES + c
    base = w * SEQ_PER_W
    x_bufs = (x_b0, x_b1, x_b2, x_b3, x_b4, x_b5, x_b6, x_b7)
    p_bufs = (p_b0, p_b1)

    def start_in(j, b, slot):
        off = b * MAXLEN + base + j * ROWS
        pltpu.async_copy(x_hbm.at[pl.ds(off, ROWS)], x_bufs[slot],
                         sem_in.at[slot])

    def wait_in(slot):
        pltpu.make_async_copy(x_hbm.at[pl.ds(0, ROWS)], x_bufs[slot],
                              sem_in.at[slot]).wait()

    def start_out(j, b, slot):
        off = b * MAXLEN + base + j * ROWS
        pltpu.async_copy(x_bufs[slot], out_hbm.at[pl.ds(off, ROWS)],
                         sem_out.at[slot])

    def wait_out(slot):
        pltpu.make_async_copy(x_bufs[slot], out_hbm.at[pl.ds(0, ROWS)],
                              sem_out.at[slot]).wait()

    def start_pos(j, pslot):
        pltpu.async_copy(pos_hbm.at[pl.ds(base + j * ROWS, ROWS)],
                         p_bufs[pslot], sem_pos.at[pslot])

    def wait_pos(pslot):
        pltpu.make_async_copy(pos_hbm.at[pl.ds(0, ROWS)], p_bufs[pslot],
                              sem_pos.at[pslot]).wait()

    def compute(g, pslot):
        xg = x_bufs[4 * g:4 * g + 4]
        pb = p_bufs[pslot]

        def rowf(r, carry):
            for cc in range(COLS16):
                sl = pl.ds(cc * LANES, LANES)
                pv = pb[r, sl]
                for xb in xg:
                    xb[r, sl] = xb[r, sl] + pv
            return carry

        lax.fori_loop(0, ROWS, rowf, 0)

    start_pos(0, 0)
    for b in range(BATCH):
        start_in(0, b, b)

    def outer(jj, carry):
        for pj in range(2):
            j = 2 * jj + pj
            g = pj
            og = 1 - g
            wait_pos(pj)
            if pj == 0:
                start_pos(j + 1, 1)
            else:
                @pl.when(jj < NJJ - 1)
                def _():
                    start_pos(j + 1, 0)
            if pj == 0:
                @pl.when(jj > 0)
                def _():
                    for b in range(BATCH):
                        wait_out(4 * og + b)
                for b in range(BATCH):
                    start_in(j + 1, b, 4 * og + b)
            else:
                for b in range(BATCH):
                    wait_out(4 * og + b)

                @pl.when(jj < NJJ - 1)
                def _():
                    for b in range(BATCH):
                        start_in(j + 1, b, 4 * og + b)
            for b in range(BATCH):
                wait_in(4 * g + b)
            compute(g, pj)
            for b in range(BATCH):
                start_out(j, b, 4 * g + b)
        return carry

    lax.fori_loop(0, NJJ, outer, 0)
    for b in range(BATCH):
        wait_out(4 + b)


@functools.partial(jax.jit)
def kernel(x, pos_table):
    mesh = plsc.VectorSubcoreMesh(
        core_axis_name="c", subcore_axis_name="s",
        num_cores=NUM_CORES, num_subcores=NUM_SUBCORES,
    )
    run = pl.kernel(
        _body,
        out_type=jax.ShapeDtypeStruct((BATCH * MAXLEN, EMBED_DIM), jnp.float32),
        mesh=mesh,
        scratch_types=[
            pltpu.VMEM((ROWS, EMBED_DIM), jnp.float32),
            pltpu.VMEM((ROWS, EMBED_DIM), jnp.float32),
            pltpu.VMEM((ROWS, EMBED_DIM), jnp.float32),
            pltpu.VMEM((ROWS, EMBED_DIM), jnp.float32),
            pltpu.VMEM((ROWS, EMBED_DIM), jnp.float32),
            pltpu.VMEM((ROWS, EMBED_DIM), jnp.float32),
            pltpu.VMEM((ROWS, EMBED_DIM), jnp.float32),
            pltpu.VMEM((ROWS, EMBED_DIM), jnp.float32),
            pltpu.VMEM((ROWS, EMBED_DIM), jnp.float32),
            pltpu.VMEM((ROWS, EMBED_DIM), jnp.float32),
            pltpu.SemaphoreType.DMA((8,)),
            pltpu.SemaphoreType.DMA((8,)),
            pltpu.SemaphoreType.DMA((2,)),
        ],
    )
    out = run(x.reshape(BATCH * MAXLEN, EMBED_DIM), pos_table)
    return out.reshape(BATCH, MAXLEN, EMBED_DIM)
